# Initial kernel scaffold; baseline (speedup 1.0000x reference)
import jax
import jax.numpy as jnp
from jax.experimental import pallas as pl


def kernel(X, C, S, edge_idx):
    raise NotImplementedError("write your pallas kernel here")


# speedup vs baseline: 8.2711x; 8.2711x over previous
import functools
import numpy as np
import jax
import jax.numpy as jnp
from jax import lax
from jax.experimental import pallas as pl
from jax.experimental.pallas import tpu as pltpu
from jax.experimental.pallas import tpu_sc as plsc

_NUM_ATOMS = np.array([5, 11, 8, 8, 6, 9, 9, 4, 10, 8, 8, 9, 8, 11, 7, 6, 7, 14, 12, 7],
                      dtype=np.int32)
_SC_ELEMS = ["C", "CCCNCNN", "CCON", "CCOO", "CS", "CCCON", "CCCOO", "", "CCNCCN",
             "CCCC", "CCCC", "CCCCN", "CCSC", "CCCCCCC", "CCC", "CO", "COC",
             "CCCCNCCCCC", "CCCCCCCO", "CCC"]
_VDW = {"C": 1.7, "N": 1.55, "O": 1.52, "S": 1.8}


def _build_vdw_table():
    R = np.zeros((20, 14), dtype=np.float32)
    for i, sc in enumerate(_SC_ELEMS):
        for j, e in enumerate("NCCO" + sc):
            R[i, j] = _VDW[e]
    return R


_VDW_R = _build_vdw_table()

_ROW = 80
_RB = 256
_CHUNK = 128
_EPS = 0.001
_CUTOFF = 0.35


def _sc_gather(table, gidx, n_rows):
    info = plsc.get_sparse_core_info()
    nc, ns = info.num_cores, info.num_subcores
    nw = nc * ns
    chunks = gidx.shape[1]
    per_w = chunks * _CHUNK
    mesh = plsc.VectorSubcoreMesh(core_axis_name="c", subcore_axis_name="s")

    @functools.partial(
        pl.kernel,
        mesh=mesh,
        compiler_params=pltpu.CompilerParams(use_tc_tiling_on_sc=False),
        out_type=jax.ShapeDtypeStruct((n_rows, _ROW), jnp.float32),
        scratch_types=[
            pltpu.VMEM((chunks, _CHUNK), jnp.int32),
            pltpu.VMEM((_CHUNK, _ROW), jnp.float32),
            pltpu.SemaphoreType.DMA,
        ],
    )
    def gather_kernel(table_hbm, gidx_hbm, out_hbm, idx_v, rows_v, sem):
        wid = lax.axis_index("s") * nc + lax.axis_index("c")
        base = wid * per_w
        pltpu.sync_copy(gidx_hbm.at[wid], idx_v)

        def body(c, carry):
            pltpu.async_copy(table_hbm.at[idx_v.at[c]], rows_v, sem).wait()
            pltpu.sync_copy(rows_v, out_hbm.at[pl.ds(base + c * _CHUNK, _CHUNK)])
            return carry

        lax.fori_loop(0, chunks, body, 0)

    return gather_kernel(table, gidx)


def _tc_body(q0_ref, q1_ref, q2_ref, qr_ref, qm_ref, g_ref, out_ref):
    k = pl.program_id(2)
    g = g_ref[0, 0]

    def tile10(v):
        return jnp.concatenate([v] * 10, axis=1)

    d2 = jnp.float32(_EPS)
    for c, q_ref in enumerate((q0_ref, q1_ref, q2_ref)):
        diff = q_ref[0] - tile10(g[:, 14 * c:14 * c + 14])
        d2 = d2 + diff * diff
    dist = jnp.sqrt(d2)
    dcut = (qr_ref[0] + tile10(g[:, 42:56])) * 0.5 + _CUTOFF
    pair = qm_ref[0] * tile10(g[:, 56:70]) * jax.nn.sigmoid(dcut - dist)
    s = jnp.sum(pair, axis=1, keepdims=True)

    rb = out_ref.shape[1]
    nvec = (pl.program_id(1) * rb
            + lax.broadcasted_iota(jnp.int32, (rb, 1), 0)).astype(jnp.float32)
    contrib = s * (g[:, 70:71] != nvec).astype(jnp.float32)

    @pl.when(k == 0)
    def _():
        out_ref[0] = contrib

    @pl.when(k != 0)
    def _():
        out_ref[0] = out_ref[0] + contrib


def _tc_compute(q0, q1, q2, qr, qm, g4, interpret=False):
    B, Kn, N, _ = g4.shape
    grid = (B, N // _RB, Kn)
    return pl.pallas_call(
        _tc_body,
        grid=grid,
        in_specs=[pl.BlockSpec((1, _RB, 140), lambda b, i, k: (b, i, 0))] * 5
        + [pl.BlockSpec((1, 1, _RB, _ROW), lambda b, i, k: (b, k, i, 0))],
        out_specs=pl.BlockSpec((1, _RB, 1), lambda b, i, k: (b, i, 0)),
        out_shape=jax.ShapeDtypeStruct((B, N, 1), jnp.float32),
        interpret=interpret,
    )(q0, q1, q2, qr, qm, g4)


def kernel(X, C, S, edge_idx):
    B, N, A, _ = X.shape
    Kn = edge_idx.shape[2]

    apr = (C > 0).astype(jnp.float32) * jnp.take(jnp.asarray(_NUM_ATOMS), S).astype(jnp.float32)
    mask = (jnp.arange(A, dtype=jnp.float32).reshape(1, 1, A) < apr[:, :, None]).astype(jnp.float32)
    radii = jnp.take(jnp.asarray(_VDW_R), S, axis=0)
    xt = jnp.transpose(X, (0, 1, 3, 2)).reshape(B, N, 3 * A)
    nloc = jnp.broadcast_to(jnp.arange(N, dtype=jnp.float32).reshape(1, N, 1), (B, N, 1))
    pad = jnp.zeros((B, N, _ROW - (3 * A + A + A + 1)), jnp.float32)
    table = jnp.concatenate([xt, radii, mask, nloc, pad], axis=-1).reshape(B * N, _ROW)

    q0 = jnp.repeat(xt[:, :, 4:14], 14, axis=-1)
    q1 = jnp.repeat(xt[:, :, 18:28], 14, axis=-1)
    q2 = jnp.repeat(xt[:, :, 32:42], 14, axis=-1)
    qr = jnp.repeat(radii[:, :, 4:14], 14, axis=-1)
    qm = jnp.repeat(mask[:, :, 4:14], 14, axis=-1)

    et = jnp.transpose(edge_idx, (0, 2, 1)).astype(jnp.int32)
    gidx = (et + (jnp.arange(B, dtype=jnp.int32) * N)[:, None, None]).reshape(-1)
    n_rows = B * Kn * N
    nw = 32
    gidx = gidx.reshape(nw, n_rows // (nw * _CHUNK), _CHUNK)

    g = _sc_gather(table, gidx, n_rows)
    g4 = g.reshape(B, Kn, N, _ROW)
    out = _tc_compute(q0, q1, q2, qr, qm, g4)
    return out[:, :, 0]


# speedup vs baseline: 21.9514x; 2.6540x over previous
import functools
import numpy as np
import jax
import jax.numpy as jnp
from jax import lax
from jax.experimental import pallas as pl
from jax.experimental.pallas import tpu as pltpu
from jax.experimental.pallas import tpu_sc as plsc

_NUM_ATOMS = np.array([5, 11, 8, 8, 6, 9, 9, 4, 10, 8, 8, 9, 8, 11, 7, 6, 7, 14, 12, 7],
                      dtype=np.int32)
_SC_ELEMS = ["C", "CCCNCNN", "CCON", "CCOO", "CS", "CCCON", "CCCOO", "", "CCNCCN",
             "CCCC", "CCCC", "CCCCN", "CCSC", "CCCCCCC", "CCC", "CO", "COC",
             "CCCCNCCCCC", "CCCCCCCO", "CCC"]
_VDW = {"C": 1.7, "N": 1.55, "O": 1.52, "S": 1.8}


def _build_vdw_table():
    R = np.zeros((20, 14), dtype=np.float32)
    for i, sc in enumerate(_SC_ELEMS):
        for j, e in enumerate("NCCO" + sc):
            R[i, j] = _VDW[e]
    return R


_VDW_R = _build_vdw_table()

_LG = 16
_RB = 256
_CHUNK = 128
_EPS = 0.001
_NEG = -1.0e4


def _sc_gather4(tables, gidx, n_rows):
    info = plsc.get_sparse_core_info()
    nc, ns = info.num_cores, info.num_subcores
    chunks = gidx.shape[1]
    per_w = chunks * _CHUNK
    mesh = plsc.VectorSubcoreMesh(core_axis_name="c", subcore_axis_name="s")
    row_t = jax.ShapeDtypeStruct((n_rows, _LG), jnp.float32)

    @functools.partial(
        pl.kernel,
        mesh=mesh,
        compiler_params=pltpu.CompilerParams(use_tc_tiling_on_sc=False),
        out_type=(row_t,) * 4,
        scratch_types=[
            pltpu.VMEM((chunks, _CHUNK), jnp.int32),
            pltpu.VMEM((_CHUNK, _LG), jnp.float32),
            pltpu.VMEM((_CHUNK, _LG), jnp.float32),
            pltpu.VMEM((_CHUNK, _LG), jnp.float32),
            pltpu.VMEM((_CHUNK, _LG), jnp.float32),
            pltpu.SemaphoreType.DMA,
            pltpu.SemaphoreType.DMA,
            pltpu.SemaphoreType.DMA,
            pltpu.SemaphoreType.DMA,
        ],
    )
    def gather_kernel(t0, t1, t2, t3, gidx_hbm, o0, o1, o2, o3,
                      idx_v, r0, r1, r2, r3, s0, s1, s2, s3):
        wid = lax.axis_index("s") * nc + lax.axis_index("c")
        base = wid * per_w
        pltpu.sync_copy(gidx_hbm.at[wid], idx_v)

        def body(c, carry):
            idx = idx_v.at[c]
            cp0 = pltpu.async_copy(t0.at[idx], r0, s0)
            cp1 = pltpu.async_copy(t1.at[idx], r1, s1)
            cp2 = pltpu.async_copy(t2.at[idx], r2, s2)
            cp3 = pltpu.async_copy(t3.at[idx], r3, s3)
            cp0.wait()
            cp1.wait()
            cp2.wait()
            cp3.wait()
            dst = pl.ds(base + c * _CHUNK, _CHUNK)
            pltpu.sync_copy(r0, o0.at[dst])
            pltpu.sync_copy(r1, o1.at[dst])
            pltpu.sync_copy(r2, o2.at[dst])
            pltpu.sync_copy(r3, o3.at[dst])
            return carry

        lax.fori_loop(0, chunks, body, 0)

    return gather_kernel(tables[0], tables[1], tables[2], tables[3], gidx)


def _tc_body(gx0_ref, gx1_ref, gx2_ref, gr_ref,
             sx0_ref, sx1_ref, sx2_ref, sr_ref, scnt_ref, out_ref):
    g0, g1, g2, gr = gx0_ref[0], gx1_ref[0], gx2_ref[0], gr_ref[0]
    s0, s1, s2, sr = sx0_ref[0], sx1_ref[0], sx2_ref[0], sr_ref[0]

    acc = jnp.zeros_like(g0)
    accs = jnp.zeros_like(s0)
    for a in range(4, 14):
        qx0 = s0[:, a:a + 1]
        qx1 = s1[:, a:a + 1]
        qx2 = s2[:, a:a + 1]
        qr = sr[:, a:a + 1]
        d2 = (qx0 - g0) ** 2 + (qx1 - g1) ** 2 + (qx2 - g2) ** 2 + _EPS
        acc = acc + (jnp.tanh(((qr + gr) - jnp.sqrt(d2)) * 0.5) + 1.0)
        d2s = (qx0 - s0) ** 2 + (qx1 - s1) ** 2 + (qx2 - s2) ** 2 + _EPS
        accs = accs + (jnp.tanh(((qr + sr) - jnp.sqrt(d2s)) * 0.5) + 1.0)

    total = jnp.sum(acc, axis=1, keepdims=True)
    self_term = jnp.sum(accs, axis=1, keepdims=True)
    out_ref[0] = 0.5 * (total - scnt_ref[0] * self_term)


def _tc_compute(gx, sx, scnt, interpret=False):
    B, N, KL = gx[0].shape
    grid = (B, N // _RB)
    gspec = pl.BlockSpec((1, _RB, KL), lambda b, i: (b, i, 0))
    sspec = pl.BlockSpec((1, _RB, _LG), lambda b, i: (b, i, 0))
    cspec = pl.BlockSpec((1, _RB, 1), lambda b, i: (b, i, 0))
    return pl.pallas_call(
        _tc_body,
        grid=grid,
        in_specs=[gspec] * 4 + [sspec] * 4 + [cspec],
        out_specs=cspec,
        out_shape=jax.ShapeDtypeStruct((B, N, 1), jnp.float32),
        interpret=interpret,
    )(*gx, *sx, scnt)


def kernel(X, C, S, edge_idx):
    B, N, A, _ = X.shape
    Kn = edge_idx.shape[2]

    apr = (C > 0).astype(jnp.float32) * jnp.take(jnp.asarray(_NUM_ATOMS), S).astype(jnp.float32)
    mask = jnp.arange(A, dtype=jnp.float32).reshape(1, 1, A) < apr[:, :, None]
    radii = jnp.take(jnp.asarray(_VDW_R), S, axis=0)
    r2 = jnp.where(mask, radii * 0.5 + 0.175, _NEG)
    pad0 = jnp.zeros((B, N, _LG - A), jnp.float32)
    padn = jnp.full((B, N, _LG - A), _NEG, jnp.float32)
    tx0 = jnp.concatenate([X[:, :, :, 0], pad0], axis=-1).reshape(B * N, _LG)
    tx1 = jnp.concatenate([X[:, :, :, 1], pad0], axis=-1).reshape(B * N, _LG)
    tx2 = jnp.concatenate([X[:, :, :, 2], pad0], axis=-1).reshape(B * N, _LG)
    tr = jnp.concatenate([r2, padn], axis=-1).reshape(B * N, _LG)
    tables = (tx0, tx1, tx2, tr)

    gidx = (edge_idx.astype(jnp.int32)
            + (jnp.arange(B, dtype=jnp.int32) * N)[:, None, None]).reshape(-1)
    n_rows = B * N * Kn
    nw = 32
    gidx3 = gidx.reshape(nw, n_rows // (nw * _CHUNK), _CHUNK)

    g = _sc_gather4(tables, gidx3, n_rows)
    gx = tuple(t.reshape(B, N, Kn * _LG) for t in g)
    sx = tuple(t.reshape(B, N, _LG) for t in tables)
    scnt = jnp.sum((edge_idx == jnp.arange(N).reshape(1, N, 1)).astype(jnp.float32),
                   axis=-1, keepdims=True)

    out = _tc_compute(gx, sx, scnt)
    return out[:, :, 0]
